# Initial kernel scaffold; baseline (speedup 1.0000x reference)
import jax
import jax.numpy as jnp
from jax.experimental import pallas as pl


def kernel(numerical_feats, categorical_feats, emb, W1, b1, W2, b2):
    raise NotImplementedError("write your pallas kernel here")


# speedup vs baseline: 7.3890x; 7.3890x over previous
import functools

import jax
import jax.numpy as jnp
from jax.experimental import pallas as pl
from jax.experimental.pallas import tpu as pltpu
from jax.experimental.pallas import tpu_sc as plsc

B = 16384
NUM_DIM = 13
N_CAT = 26
VOCAB = 100000
EMB = 16
OUT = 128
TOTAL = NUM_DIM + N_CAT * EMB
HID = TOTAL * 2

_GATHER_WINDOW = 128
_N_IDX = B * N_CAT

_BM = 2048


def _gather_rows(table2d, gidx):
    mesh = plsc.VectorSubcoreMesh(core_axis_name="core", subcore_axis_name="subcore")
    gidx2d = gidx.reshape(1, _N_IDX)

    @functools.partial(
        pl.kernel,
        out_type=jax.ShapeDtypeStruct((_N_IDX, EMB), jnp.float32),
        mesh=mesh,
        compiler_params=pltpu.CompilerParams(use_tc_tiling_on_sc=False),
    )
    def gather_kernel(x_hbm, i_hbm, o_hbm):
        def body(i_vmem, o_vmem):
            pltpu.sync_copy(x_hbm.at[i_vmem.at[0]], o_vmem)

        pltpu.emit_pipeline(
            body,
            grid=(_N_IDX // _GATHER_WINDOW,),
            in_specs=[pl.BlockSpec((1, _GATHER_WINDOW), index_map=lambda i: (0, i))],
            out_specs=[pl.BlockSpec((_GATHER_WINDOW, EMB), index_map=lambda i: (i, 0))],
            core_axis_name=("core", "subcore"),
            dimension_semantics=(pltpu.PARALLEL,),
        )(i_hbm, o_hbm)

    return gather_kernel(table2d, gidx2d)


def _mlp_body(num_ref, cat_ref, w1n_ref, w1c_ref, b1_ref, w2_ref, b2_ref, out_ref):
    h = jnp.dot(cat_ref[...], w1c_ref[...], preferred_element_type=jnp.float32)
    h += jnp.dot(num_ref[...], w1n_ref[...], preferred_element_type=jnp.float32)
    h = jnp.maximum(h + b1_ref[...], 0.0)
    out_ref[...] = (
        jnp.dot(h, w2_ref[...], preferred_element_type=jnp.float32) + b2_ref[...]
    )


def _mlp(num, cat_flat, w1n, w1c, b1, w2, b2):
    grid = (B // _BM,)
    return pl.pallas_call(
        _mlp_body,
        grid=grid,
        in_specs=[
            pl.BlockSpec((_BM, NUM_DIM), lambda i: (i, 0)),
            pl.BlockSpec((_BM, N_CAT * EMB), lambda i: (i, 0)),
            pl.BlockSpec((NUM_DIM, HID), lambda i: (0, 0)),
            pl.BlockSpec((N_CAT * EMB, HID), lambda i: (0, 0)),
            pl.BlockSpec((1, HID), lambda i: (0, 0)),
            pl.BlockSpec((HID, OUT), lambda i: (0, 0)),
            pl.BlockSpec((1, OUT), lambda i: (0, 0)),
        ],
        out_specs=pl.BlockSpec((_BM, OUT), lambda i: (i, 0)),
        out_shape=jax.ShapeDtypeStruct((B, OUT), jnp.float32),
    )(num, cat_flat, w1n, w1c, b1, w2, b2)


def kernel(numerical_feats, categorical_feats, emb, W1, b1, W2, b2):
    table2d = emb.reshape(N_CAT * VOCAB, EMB)
    offsets = (jnp.arange(N_CAT, dtype=jnp.int32) * VOCAB)[None, :]
    gidx = (categorical_feats + offsets).reshape(-1)
    rows = _gather_rows(table2d, gidx)
    cat_flat = rows.reshape(B, N_CAT * EMB)
    w1n = W1[:NUM_DIM]
    w1c = W1[NUM_DIM:]
    return _mlp(
        numerical_feats,
        cat_flat,
        w1n,
        w1c,
        b1.reshape(1, HID),
        W2,
        b2.reshape(1, OUT),
    )


# speedup vs baseline: 14.8386x; 2.0082x over previous
import functools

import jax
import jax.numpy as jnp
from jax import lax
from jax.experimental import pallas as pl
from jax.experimental.pallas import tpu as pltpu
from jax.experimental.pallas import tpu_sc as plsc

B = 16384
NUM_DIM = 13
N_CAT = 26
VOCAB = 100000
EMB = 16
OUT = 128
TOTAL = NUM_DIM + N_CAT * EMB
HID = TOTAL * 2

NJ = N_CAT * EMB
_GW = 128
_NW = 32
_JPW = NJ // _NW
_NWIN = B // _GW

_BM = 2048


def _gather_t(tabT, catT):
    mesh = plsc.VectorSubcoreMesh(core_axis_name="core", subcore_axis_name="subcore")

    @functools.partial(
        pl.kernel,
        out_type=jax.ShapeDtypeStruct((NJ, B), jnp.float32),
        mesh=mesh,
        compiler_params=pltpu.CompilerParams(use_tc_tiling_on_sc=False),
        scratch_types=[
            pltpu.VMEM((B,), jnp.int32),
            pltpu.VMEM((B,), jnp.float32),
            pltpu.SemaphoreType.DMA,
        ],
    )
    def k(t_hbm, i_hbm, o_hbm, idx_v, out_v, sem):
        wid = lax.axis_index("subcore") * 2 + lax.axis_index("core")

        @pl.loop(0, _JPW)
        def _(jj):
            j = wid * _JPW + jj
            f = j // EMB
            pltpu.sync_copy(i_hbm.at[f], idx_v)

            @pl.loop(0, _NWIN)
            def _(w):
                pltpu.async_copy(
                    t_hbm.at[j].at[idx_v.at[pl.ds(w * _GW, _GW)]],
                    out_v.at[pl.ds(w * _GW, _GW)],
                    sem,
                )

            @pl.loop(0, _NWIN)
            def _(w):
                pltpu.make_async_copy(
                    t_hbm.at[j].at[idx_v.at[pl.ds(w * _GW, _GW)]],
                    out_v.at[pl.ds(w * _GW, _GW)],
                    sem,
                ).wait()

            pltpu.sync_copy(out_v, o_hbm.at[j])

    return k(tabT, catT)


def _mlp_body(numT_ref, catT_ref, w1n_ref, w1c_ref, b1_ref, w2_ref, b2_ref, out_ref):
    cdims = (((0,), (0,)), ((), ()))
    h = lax.dot_general(
        catT_ref[...], w1c_ref[...], cdims, preferred_element_type=jnp.float32
    )
    h += lax.dot_general(
        numT_ref[...], w1n_ref[...], cdims, preferred_element_type=jnp.float32
    )
    h = jnp.maximum(h + b1_ref[...], 0.0)
    out_ref[...] = (
        jnp.dot(h, w2_ref[...], preferred_element_type=jnp.float32) + b2_ref[...]
    )


def _mlp(numT, catT, w1n, w1c, b1, w2, b2):
    grid = (B // _BM,)
    return pl.pallas_call(
        _mlp_body,
        grid=grid,
        in_specs=[
            pl.BlockSpec((NUM_DIM, _BM), lambda i: (0, i)),
            pl.BlockSpec((NJ, _BM), lambda i: (0, i)),
            pl.BlockSpec((NUM_DIM, HID), lambda i: (0, 0)),
            pl.BlockSpec((NJ, HID), lambda i: (0, 0)),
            pl.BlockSpec((1, HID), lambda i: (0, 0)),
            pl.BlockSpec((HID, OUT), lambda i: (0, 0)),
            pl.BlockSpec((1, OUT), lambda i: (0, 0)),
        ],
        out_specs=pl.BlockSpec((_BM, OUT), lambda i: (i, 0)),
        out_shape=jax.ShapeDtypeStruct((B, OUT), jnp.float32),
    )(numT, catT, w1n, w1c, b1, w2, b2)


def kernel(numerical_feats, categorical_feats, emb, W1, b1, W2, b2):
    tabT = emb.transpose(0, 2, 1).reshape(NJ, VOCAB)
    catT = categorical_feats.T
    outT = _gather_t(tabT, catT)
    numT = numerical_feats.T
    w1n = W1[:NUM_DIM]
    w1c = W1[NUM_DIM:]
    return _mlp(
        numT,
        outT,
        w1n,
        w1c,
        b1.reshape(1, HID),
        W2,
        b2.reshape(1, OUT),
    )


# speedup vs baseline: 31.8090x; 2.1437x over previous
import functools

import jax
import jax.numpy as jnp
from jax import lax
from jax.experimental import pallas as pl
from jax.experimental.pallas import tpu as pltpu
from jax.experimental.pallas import tpu_sc as plsc

B = 16384
NUM_DIM = 13
N_CAT = 26
VOCAB = 100000
EMB = 16
OUT = 128
TOTAL = NUM_DIM + N_CAT * EMB
HID = TOTAL * 2

NJ = N_CAT * EMB
_NW = 32
_JPW = NJ // _NW
_CHUNK = 2048
_NCHUNK = B // _CHUNK
_L = 16

_BM = 2048


def _gather_t(tabT, catT):
    mesh = plsc.VectorSubcoreMesh(core_axis_name="core", subcore_axis_name="subcore")

    @functools.partial(
        pl.kernel,
        out_type=jax.ShapeDtypeStruct((NJ, B), jnp.float32),
        mesh=mesh,
        compiler_params=pltpu.CompilerParams(
            use_tc_tiling_on_sc=True, needs_layout_passes=False
        ),
        scratch_types=[
            pltpu.VMEM((VOCAB,), jnp.float32),
            pltpu.VMEM((B,), jnp.int32),
            pltpu.VMEM((_CHUNK,), jnp.float32),
        ],
    )
    def k(t_hbm, i_hbm, o_hbm, row_v, idx_v, outw_v):
        wid = lax.axis_index("subcore") * 2 + lax.axis_index("core")

        @pl.loop(0, _JPW)
        def _(jj):
            j = wid * _JPW + jj
            f = j // EMB
            pltpu.sync_copy(i_hbm.at[f], idx_v)
            pltpu.sync_copy(t_hbm.at[j], row_v)

            @pl.loop(0, _NCHUNK)
            def _(c):
                @pl.loop(0, _CHUNK // _L)
                def _(kk):
                    idx16 = idx_v[pl.ds(c * _CHUNK + kk * _L, _L)]
                    outw_v[pl.ds(kk * _L, _L)] = plsc.load_gather(row_v, [idx16])

                pltpu.sync_copy(outw_v, o_hbm.at[j].at[pl.ds(c * _CHUNK, _CHUNK)])

    return k(tabT, catT)


def _mlp_body(numT_ref, catT_ref, w1n_ref, w1c_ref, b1_ref, w2_ref, b2_ref, out_ref):
    cdims = (((0,), (0,)), ((), ()))
    h = lax.dot_general(
        catT_ref[...], w1c_ref[...], cdims, preferred_element_type=jnp.float32
    )
    h += lax.dot_general(
        numT_ref[...], w1n_ref[...], cdims, preferred_element_type=jnp.float32
    )
    h = jnp.maximum(h + b1_ref[...], 0.0)
    out_ref[...] = (
        jnp.dot(h, w2_ref[...], preferred_element_type=jnp.float32) + b2_ref[...]
    )


def _mlp(numT, catT, w1n, w1c, b1, w2, b2):
    grid = (B // _BM,)
    return pl.pallas_call(
        _mlp_body,
        grid=grid,
        in_specs=[
            pl.BlockSpec((NUM_DIM, _BM), lambda i: (0, i)),
            pl.BlockSpec((NJ, _BM), lambda i: (0, i)),
            pl.BlockSpec((NUM_DIM, HID), lambda i: (0, 0)),
            pl.BlockSpec((NJ, HID), lambda i: (0, 0)),
            pl.BlockSpec((1, HID), lambda i: (0, 0)),
            pl.BlockSpec((HID, OUT), lambda i: (0, 0)),
            pl.BlockSpec((1, OUT), lambda i: (0, 0)),
        ],
        out_specs=pl.BlockSpec((_BM, OUT), lambda i: (i, 0)),
        out_shape=jax.ShapeDtypeStruct((B, OUT), jnp.float32),
    )(numT, catT, w1n, w1c, b1, w2, b2)


def kernel(numerical_feats, categorical_feats, emb, W1, b1, W2, b2):
    tabT = emb.transpose(0, 2, 1).reshape(NJ, VOCAB)
    catT = categorical_feats.T
    outT = _gather_t(tabT, catT)
    numT = numerical_feats.T
    w1n = W1[:NUM_DIM]
    w1c = W1[NUM_DIM:]
    return _mlp(
        numT,
        outT,
        w1n,
        w1c,
        b1.reshape(1, HID),
        W2,
        b2.reshape(1, OUT),
    )


# speedup vs baseline: 45.7289x; 1.4376x over previous
import functools

import jax
import jax.numpy as jnp
from jax import lax
from jax.experimental import pallas as pl
from jax.experimental.pallas import tpu as pltpu
from jax.experimental.pallas import tpu_sc as plsc

B = 16384
NUM_DIM = 13
N_CAT = 26
VOCAB = 100000
EMB = 16
OUT = 128
TOTAL = NUM_DIM + N_CAT * EMB
HID = TOTAL * 2

NJ = N_CAT * EMB
_NW = 32
_JPW = NJ // _NW
_HALF = B // 2
_L = 16
_UNROLL = 8

_BM = 2048


def _gather_t(tabT, catT):
    mesh = plsc.VectorSubcoreMesh(core_axis_name="core", subcore_axis_name="subcore")

    @functools.partial(
        pl.kernel,
        out_type=jax.ShapeDtypeStruct((NJ, B), jnp.float32),
        mesh=mesh,
        compiler_params=pltpu.CompilerParams(
            use_tc_tiling_on_sc=True, needs_layout_passes=False
        ),
        scratch_types=[
            pltpu.VMEM((VOCAB,), jnp.float32),
            pltpu.VMEM((B,), jnp.int32),
            pltpu.VMEM((_HALF,), jnp.float32),
            pltpu.SemaphoreType.DMA,
            pltpu.SemaphoreType.DMA,
        ],
    )
    def k(t_hbm, i_hbm, o_hbm, row_v, idx_v, outh_v, lsem, wsem):
        wid = lax.axis_index("subcore") * 2 + lax.axis_index("core")

        @pl.loop(0, _JPW)
        def _(jj):
            j = wid * _JPW + jj
            f = j // EMB
            a_idx = pltpu.async_copy(i_hbm.at[f], idx_v, lsem)
            a_row = pltpu.async_copy(t_hbm.at[j], row_v, lsem)
            a_idx.wait()
            a_row.wait()

            for h in range(2):
                if h == 1:
                    pltpu.make_async_copy(
                        outh_v, o_hbm.at[0].at[pl.ds(0, _HALF)], wsem
                    ).wait()
                else:

                    @pl.when(jj > 0)
                    def _():
                        pltpu.make_async_copy(
                            outh_v, o_hbm.at[0].at[pl.ds(0, _HALF)], wsem
                        ).wait()

                @pl.loop(0, _HALF // (_L * _UNROLL))
                def _(kk):
                    base = kk * (_L * _UNROLL)
                    for u in range(_UNROLL):
                        o = base + u * _L
                        idx16 = idx_v[pl.ds(h * _HALF + o, _L)]
                        outh_v[pl.ds(o, _L)] = plsc.load_gather(row_v, [idx16])

                pltpu.async_copy(
                    outh_v, o_hbm.at[j].at[pl.ds(h * _HALF, _HALF)], wsem
                )

        pltpu.make_async_copy(
            outh_v, o_hbm.at[0].at[pl.ds(0, _HALF)], wsem
        ).wait()

    return k(tabT, catT)


def _mlp_body(numT_ref, catT_ref, w1n_ref, w1c_ref, b1_ref, w2_ref, b2_ref, out_ref):
    cdims = (((0,), (0,)), ((), ()))
    h = lax.dot_general(
        catT_ref[...], w1c_ref[...], cdims, preferred_element_type=jnp.float32
    )
    h += lax.dot_general(
        numT_ref[...], w1n_ref[...], cdims, preferred_element_type=jnp.float32
    )
    h = jnp.maximum(h + b1_ref[...], 0.0)
    out_ref[...] = (
        jnp.dot(h, w2_ref[...], preferred_element_type=jnp.float32) + b2_ref[...]
    )


def _mlp(numT, catT, w1n, w1c, b1, w2, b2):
    grid = (B // _BM,)
    return pl.pallas_call(
        _mlp_body,
        grid=grid,
        in_specs=[
            pl.BlockSpec((NUM_DIM, _BM), lambda i: (0, i)),
            pl.BlockSpec((NJ, _BM), lambda i: (0, i)),
            pl.BlockSpec((NUM_DIM, HID), lambda i: (0, 0)),
            pl.BlockSpec((NJ, HID), lambda i: (0, 0)),
            pl.BlockSpec((1, HID), lambda i: (0, 0)),
            pl.BlockSpec((HID, OUT), lambda i: (0, 0)),
            pl.BlockSpec((1, OUT), lambda i: (0, 0)),
        ],
        out_specs=pl.BlockSpec((_BM, OUT), lambda i: (i, 0)),
        out_shape=jax.ShapeDtypeStruct((B, OUT), jnp.float32),
    )(numT, catT, w1n, w1c, b1, w2, b2)


def kernel(numerical_feats, categorical_feats, emb, W1, b1, W2, b2):
    tabT = emb.transpose(0, 2, 1).reshape(NJ, VOCAB)
    catT = categorical_feats.T
    outT = _gather_t(tabT, catT)
    numT = numerical_feats.T
    w1n = W1[:NUM_DIM]
    w1c = W1[NUM_DIM:]
    return _mlp(
        numT,
        outT,
        w1n,
        w1c,
        b1.reshape(1, HID),
        W2,
        b2.reshape(1, OUT),
    )
